# baseline (speedup 1.0000x reference)
import jax
import jax.numpy as jnp
from jax import lax
from jax.experimental import pallas as pl
from jax.experimental.pallas import tpu as pltpu
from jax.experimental.pallas import tpu_sc as plsc

_V = 1000000
_D = 64
_BATCH = 4096
_S = 200
_NC = 2
_NS = 16
_NW = _NC * _NS

_NTC_FULL = _V // 128
_TPW = 244
_REM = _NTC_FULL - _TPW * _NW
_VPAD = (_NTC_FULL + 1) * 128

_NCH = _BATCH // _NW
_BPW = _NCH * _S
_NBUF = 4


def _transpose_body(tt_hbm, tpad_hbm, stage, outb, sems_i, sems_o, tail_v):
    wid = lax.axis_index("s") * _NC + lax.axis_index("c")
    lo = wid * _TPW

    def start_in(u, b):
        pltpu.async_copy(
            tt_hbm.at[:, pl.ds(128 * (lo + u), 128)], stage[b], sems_i[b])

    def wait_in(u, b):
        pltpu.make_async_copy(
            tt_hbm.at[:, pl.ds(128 * (lo + u), 128)], stage[b], sems_i[b]).wait()

    def start_out(u, b):
        pltpu.async_copy(
            outb[b], tpad_hbm.at[pl.ds(128 * (lo + u), 128)], sems_o[b])

    def wait_out(u, b):
        pltpu.make_async_copy(
            outb[b], tpad_hbm.at[pl.ds(128 * (lo + u), 128)], sems_o[b]).wait()

    def transpose(b, nrows):
        def do_row(j, carry):
            jj = jnp.full((16,), 0, jnp.int32) + j
            for q in range(4):
                dd = lax.iota(jnp.int32, 16) + 16 * q
                vals = plsc.load_gather(stage[b], [dd, jj])
                outb[b][j, pl.ds(16 * q, 16)] = vals
            return carry
        lax.fori_loop(0, nrows, do_row, 0)

    start_in(0, 0)
    start_in(1, 1)
    wait_in(0, 0)
    transpose(0, 128)
    start_out(0, 0)
    start_in(2, 0)
    wait_in(1, 1)
    transpose(1, 128)
    start_out(1, 1)

    def body(t, carry):
        for j in range(2):
            u = 2 + t * 2 + j
            b = j
            start_in(u + 1, 1 - b)
            wait_in(u, b)
            wait_out(u - 2, b)
            transpose(b, 128)
            start_out(u, b)
        return carry

    lax.fori_loop(0, (_TPW - 4) // 2, body, 0)

    start_in(243, 1)
    wait_in(242, 0)
    wait_out(240, 0)
    transpose(0, 128)
    start_out(242, 0)
    wait_in(243, 1)
    wait_out(241, 1)
    transpose(1, 128)
    start_out(243, 1)
    wait_out(242, 0)
    wait_out(243, 1)

    @pl.when(wid < _REM)
    def _leftover():
        tc = _NTC_FULL - _REM + wid
        pltpu.sync_copy(tt_hbm.at[:, pl.ds(128 * tc, 128)], stage[0])
        transpose(0, 128)
        pltpu.sync_copy(outb[0], tpad_hbm.at[pl.ds(128 * tc, 128)])

    @pl.when(wid == _NW - 1)
    def _tail():
        pltpu.sync_copy(tt_hbm.at[:, pl.ds(128 * _NTC_FULL, 64)], tail_v)
        def do_row(j, carry):
            jj = jnp.full((16,), 0, jnp.int32) + j
            for q in range(4):
                dd = lax.iota(jnp.int32, 16) + 16 * q
                vals = plsc.load_gather(tail_v, [dd, jj])
                outb[0][j, pl.ds(16 * q, 16)] = vals
            return carry
        lax.fori_loop(0, 64, do_row, 0)
        pltpu.sync_copy(
            outb[0].at[pl.ds(0, 64)], tpad_hbm.at[pl.ds(128 * _NTC_FULL, 64)])


def _gather_body(x_hbm, tpad_hbm, out_hbm, idx_v, rows, sems_g, sems_w):
    wid = lax.axis_index("s") * _NC + lax.axis_index("c")
    base = wid * _NCH

    pltpu.sync_copy(x_hbm.at[pl.ds(base * _S, _BPW)], idx_v)

    def issue_gather(g, b):
        pltpu.async_copy(
            tpad_hbm.at[idx_v.at[pl.ds(g * _S, _S)]], rows[b], sems_g[b])

    def wait_gather(g, b):
        pltpu.make_async_copy(
            tpad_hbm.at[idx_v.at[pl.ds(g * _S, _S)]], rows[b], sems_g[b]).wait()

    def issue_wb(g, b):
        pltpu.async_copy(rows[b], out_hbm.at[base + g], sems_w[b])

    def wait_wb(g, b):
        pltpu.make_async_copy(rows[b], out_hbm.at[base + g], sems_w[b]).wait()

    for b in range(_NBUF):
        issue_gather(b, b)

    wait_gather(0, 0)
    issue_wb(0, 0)

    def body(t, carry):
        for j in range(_NBUF):
            g = 1 + t * _NBUF + j
            bp = j
            b = (1 + j) % _NBUF
            wait_wb(g - 1, bp)
            issue_gather(g - 1 + _NBUF, bp)
            wait_gather(g, b)
            issue_wb(g, b)
        return carry

    lax.fori_loop(0, (_NCH - _NBUF) // _NBUF, body, 0)

    for g in range(_NCH - _NBUF + 1, _NCH):
        wait_wb(g - 1, (g - 1) % _NBUF)
        wait_gather(g, g % _NBUF)
        issue_wb(g, g % _NBUF)

    wait_wb(_NCH - 1, (_NCH - 1) % _NBUF)


@jax.jit
def kernel(x, table):
    mesh = plsc.VectorSubcoreMesh(core_axis_name="c", subcore_axis_name="s")
    tpad = pl.kernel(
        _transpose_body,
        out_type=jax.ShapeDtypeStruct((_VPAD, 128), jnp.float32),
        mesh=mesh,
        scratch_types=[
            [pltpu.VMEM((_D, 128), jnp.float32) for _ in range(2)],
            [pltpu.VMEM((128, 128), jnp.float32) for _ in range(2)],
            [pltpu.SemaphoreType.DMA for _ in range(2)],
            [pltpu.SemaphoreType.DMA for _ in range(2)],
            pltpu.VMEM((_D, 64), jnp.float32),
        ],
        compiler_params=pltpu.CompilerParams(use_tc_tiling_on_sc=True,
                                             needs_layout_passes=False),
    )(table.T)
    out = pl.kernel(
        _gather_body,
        out_type=jax.ShapeDtypeStruct((_BATCH, _S, 128), jnp.float32),
        mesh=mesh,
        scratch_types=[
            pltpu.VMEM((_BPW,), jnp.int32),
            [pltpu.VMEM((_S, 128), jnp.float32) for _ in range(_NBUF)],
            [pltpu.SemaphoreType.DMA for _ in range(_NBUF)],
            [pltpu.SemaphoreType.DMA for _ in range(_NBUF)],
        ],
        compiler_params=pltpu.CompilerParams(use_tc_tiling_on_sc=True),
    )(x.reshape(-1).astype(jnp.int32), tpad)
    return out[:, :, :_D]


# speedup vs baseline: 1.1627x; 1.1627x over previous
import jax
import jax.numpy as jnp
from jax import lax
from jax.experimental import pallas as pl
from jax.experimental.pallas import tpu as pltpu
from jax.experimental.pallas import tpu_sc as plsc

_V = 1000000
_D = 64
_BATCH = 4096
_S = 200
_NC = 2
_NS = 16
_NW = _NC * _NS

_NTC_FULL = _V // 128
_TPW = 244
_REM = _NTC_FULL - _TPW * _NW
_VPAD = (_NTC_FULL + 1) * 128

_NCH = _BATCH // _NW
_BPW = _NCH * _S
_NBUF = 4


def _transpose_body(tt_hbm, tpad_hbm, stage, outb, sems_i, sems_o, tail_v):
    wid = lax.axis_index("s") * _NC + lax.axis_index("c")
    lo = wid * _TPW

    def start_in(u, b):
        pltpu.async_copy(
            tt_hbm.at[:, pl.ds(128 * (lo + u), 128)], stage[b], sems_i[b])

    def wait_in(u, b):
        pltpu.make_async_copy(
            tt_hbm.at[:, pl.ds(128 * (lo + u), 128)], stage[b], sems_i[b]).wait()

    def start_out(u, b):
        pltpu.async_copy(
            outb[b].at[:, pl.ds(0, 128)],
            tpad_hbm.at[pl.ds(128 * (lo + u), 128)], sems_o[b])

    def wait_out(u, b):
        pltpu.make_async_copy(
            outb[b].at[:, pl.ds(0, 128)],
            tpad_hbm.at[pl.ds(128 * (lo + u), 128)], sems_o[b]).wait()

    jbs = [lax.iota(jnp.int32, 16) + 16 * jb for jb in range(8)]

    def transpose(b, njb):
        def do_d(d, carry):
            dd = jnp.full((16,), 0, jnp.int32) + d
            for jb in range(njb):
                vals = stage[b][d, pl.ds(16 * jb, 16)]
                plsc.store_scatter(outb[b], [jbs[jb], dd], vals)
            return carry
        lax.fori_loop(0, _D, do_d, 0)

    start_in(0, 0)
    start_in(1, 1)
    wait_in(0, 0)
    transpose(0, 8)
    start_out(0, 0)
    start_in(2, 0)
    wait_in(1, 1)
    transpose(1, 8)
    start_out(1, 1)

    def body(t, carry):
        for j in range(2):
            u = 2 + t * 2 + j
            b = j
            start_in(u + 1, 1 - b)
            wait_in(u, b)
            wait_out(u - 2, b)
            transpose(b, 8)
            start_out(u, b)
        return carry

    lax.fori_loop(0, (_TPW - 4) // 2, body, 0)

    start_in(243, 1)
    wait_in(242, 0)
    wait_out(240, 0)
    transpose(0, 8)
    start_out(242, 0)
    wait_in(243, 1)
    wait_out(241, 1)
    transpose(1, 8)
    start_out(243, 1)
    wait_out(242, 0)
    wait_out(243, 1)

    @pl.when(wid < _REM)
    def _leftover():
        tc = _NTC_FULL - _REM + wid
        pltpu.sync_copy(tt_hbm.at[:, pl.ds(128 * tc, 128)], stage[0])
        transpose(0, 8)
        pltpu.sync_copy(
            outb[0].at[:, pl.ds(0, 128)], tpad_hbm.at[pl.ds(128 * tc, 128)])

    @pl.when(wid == _NW - 1)
    def _tail():
        pltpu.sync_copy(tt_hbm.at[:, pl.ds(128 * _NTC_FULL, 64)], tail_v)
        def do_d(d, carry):
            dd = jnp.full((16,), 0, jnp.int32) + d
            for jb in range(4):
                vals = tail_v[d, pl.ds(16 * jb, 16)]
                plsc.store_scatter(outb[0], [jbs[jb], dd], vals)
            return carry
        lax.fori_loop(0, _D, do_d, 0)
        pltpu.sync_copy(
            outb[0].at[pl.ds(0, 64), pl.ds(0, 128)],
            tpad_hbm.at[pl.ds(128 * _NTC_FULL, 64)])


def _gather_body(x_hbm, tpad_hbm, out_hbm, idx_v, rows, sems_g, sems_w):
    wid = lax.axis_index("s") * _NC + lax.axis_index("c")
    base = wid * _NCH

    pltpu.sync_copy(x_hbm.at[pl.ds(base * _S, _BPW)], idx_v)

    def issue_gather(g, b):
        pltpu.async_copy(
            tpad_hbm.at[idx_v.at[pl.ds(g * _S, _S)]], rows[b], sems_g[b])

    def wait_gather(g, b):
        pltpu.make_async_copy(
            tpad_hbm.at[idx_v.at[pl.ds(g * _S, _S)]], rows[b], sems_g[b]).wait()

    def issue_wb(g, b):
        pltpu.async_copy(rows[b], out_hbm.at[base + g], sems_w[b])

    def wait_wb(g, b):
        pltpu.make_async_copy(rows[b], out_hbm.at[base + g], sems_w[b]).wait()

    for b in range(_NBUF):
        issue_gather(b, b)

    wait_gather(0, 0)
    issue_wb(0, 0)

    def body(t, carry):
        for j in range(_NBUF):
            g = 1 + t * _NBUF + j
            bp = j
            b = (1 + j) % _NBUF
            wait_wb(g - 1, bp)
            issue_gather(g - 1 + _NBUF, bp)
            wait_gather(g, b)
            issue_wb(g, b)
        return carry

    lax.fori_loop(0, (_NCH - _NBUF) // _NBUF, body, 0)

    for g in range(_NCH - _NBUF + 1, _NCH):
        wait_wb(g - 1, (g - 1) % _NBUF)
        wait_gather(g, g % _NBUF)
        issue_wb(g, g % _NBUF)

    wait_wb(_NCH - 1, (_NCH - 1) % _NBUF)


@jax.jit
def kernel(x, table):
    mesh = plsc.VectorSubcoreMesh(core_axis_name="c", subcore_axis_name="s")
    tpad = pl.kernel(
        _transpose_body,
        out_type=jax.ShapeDtypeStruct((_VPAD, 128), jnp.float32),
        mesh=mesh,
        scratch_types=[
            [pltpu.VMEM((_D, 128), jnp.float32) for _ in range(2)],
            [pltpu.VMEM((128, 133), jnp.float32) for _ in range(2)],
            [pltpu.SemaphoreType.DMA for _ in range(2)],
            [pltpu.SemaphoreType.DMA for _ in range(2)],
            pltpu.VMEM((_D, 64), jnp.float32),
        ],
        compiler_params=pltpu.CompilerParams(use_tc_tiling_on_sc=True,
                                             needs_layout_passes=False),
    )(table.T)
    out = pl.kernel(
        _gather_body,
        out_type=jax.ShapeDtypeStruct((_BATCH, _S, 128), jnp.float32),
        mesh=mesh,
        scratch_types=[
            pltpu.VMEM((_BPW,), jnp.int32),
            [pltpu.VMEM((_S, 128), jnp.float32) for _ in range(_NBUF)],
            [pltpu.SemaphoreType.DMA for _ in range(_NBUF)],
            [pltpu.SemaphoreType.DMA for _ in range(_NBUF)],
        ],
        compiler_params=pltpu.CompilerParams(use_tc_tiling_on_sc=True),
    )(x.reshape(-1).astype(jnp.int32), tpad)
    return out[:, :, :_D]


# speedup vs baseline: 1.1688x; 1.0053x over previous
import jax
import jax.numpy as jnp
from jax import lax
from jax.experimental import pallas as pl
from jax.experimental.pallas import tpu as pltpu
from jax.experimental.pallas import tpu_sc as plsc

_V = 1000000
_D = 64
_BATCH = 4096
_S = 200
_NC = 2
_NS = 16
_NW = _NC * _NS

_NTC_FULL = _V // 128
_TPW = 244
_REM = _NTC_FULL - _TPW * _NW
_VPAD = (_NTC_FULL + 1) * 128

_NCH = _BATCH // _NW
_BPW = _NCH * _S
_NBUF = 4


def _transpose_body(tt_hbm, tpad_hbm, stage, outb, sems_i, sems_o, tail_v):
    wid = lax.axis_index("s") * _NC + lax.axis_index("c")
    lo = wid * _TPW

    def start_in(u, b):
        pltpu.async_copy(
            tt_hbm.at[:, pl.ds(128 * (lo + u), 128)], stage[b], sems_i[b])

    def wait_in(u, b):
        pltpu.make_async_copy(
            tt_hbm.at[:, pl.ds(128 * (lo + u), 128)], stage[b], sems_i[b]).wait()

    def start_out(u, b):
        pltpu.async_copy(
            outb[b].at[:, pl.ds(0, 128)],
            tpad_hbm.at[pl.ds(128 * (lo + u), 128)], sems_o[b])

    def wait_out(u, b):
        pltpu.make_async_copy(
            outb[b].at[:, pl.ds(0, 128)],
            tpad_hbm.at[pl.ds(128 * (lo + u), 128)], sems_o[b]).wait()

    jbs = [lax.iota(jnp.int32, 16) + 16 * jb for jb in range(8)]

    def transpose(b, njb):
        def do_d(d2, carry):
            for h in range(2):
                d = d2 * 2 + h
                dd = jnp.full((16,), 0, jnp.int32) + d
                vals = [stage[b][d, pl.ds(16 * jb, 16)] for jb in range(njb)]
                for jb in range(njb):
                    plsc.store_scatter(outb[b], [jbs[jb], dd], vals[jb])
            return carry
        lax.fori_loop(0, _D // 2, do_d, 0)

    start_in(0, 0)
    start_in(1, 1)
    wait_in(0, 0)
    transpose(0, 8)
    start_out(0, 0)
    start_in(2, 0)
    wait_in(1, 1)
    transpose(1, 8)
    start_out(1, 1)

    def body(t, carry):
        for j in range(2):
            u = 2 + t * 2 + j
            b = j
            start_in(u + 1, 1 - b)
            wait_in(u, b)
            wait_out(u - 2, b)
            transpose(b, 8)
            start_out(u, b)
        return carry

    lax.fori_loop(0, (_TPW - 4) // 2, body, 0)

    start_in(243, 1)
    wait_in(242, 0)
    wait_out(240, 0)
    transpose(0, 8)
    start_out(242, 0)
    wait_in(243, 1)
    wait_out(241, 1)
    transpose(1, 8)
    start_out(243, 1)
    wait_out(242, 0)
    wait_out(243, 1)

    @pl.when(wid < _REM)
    def _leftover():
        tc = _NTC_FULL - _REM + wid
        pltpu.sync_copy(tt_hbm.at[:, pl.ds(128 * tc, 128)], stage[0])
        transpose(0, 8)
        pltpu.sync_copy(
            outb[0].at[:, pl.ds(0, 128)], tpad_hbm.at[pl.ds(128 * tc, 128)])

    @pl.when(wid == _NW - 1)
    def _tail():
        pltpu.sync_copy(tt_hbm.at[:, pl.ds(128 * _NTC_FULL, 64)], tail_v)
        def do_d(d, carry):
            dd = jnp.full((16,), 0, jnp.int32) + d
            vals = [tail_v[d, pl.ds(16 * jb, 16)] for jb in range(4)]
            for jb in range(4):
                plsc.store_scatter(outb[0], [jbs[jb], dd], vals[jb])
            return carry
        lax.fori_loop(0, _D, do_d, 0)
        pltpu.sync_copy(
            outb[0].at[pl.ds(0, 64), pl.ds(0, 128)],
            tpad_hbm.at[pl.ds(128 * _NTC_FULL, 64)])


def _gather_body(x_hbm, tpad_hbm, out_hbm, idx_v, rows, sems_g, sems_w):
    wid = lax.axis_index("s") * _NC + lax.axis_index("c")
    base = wid * _NCH

    pltpu.sync_copy(x_hbm.at[pl.ds(base * _S, _BPW)], idx_v)

    def issue_gather(g, b):
        pltpu.async_copy(
            tpad_hbm.at[idx_v.at[pl.ds(g * _S, _S)]], rows[b], sems_g[b])

    def wait_gather(g, b):
        pltpu.make_async_copy(
            tpad_hbm.at[idx_v.at[pl.ds(g * _S, _S)]], rows[b], sems_g[b]).wait()

    def issue_wb(g, b):
        pltpu.async_copy(rows[b], out_hbm.at[base + g], sems_w[b])

    def wait_wb(g, b):
        pltpu.make_async_copy(rows[b], out_hbm.at[base + g], sems_w[b]).wait()

    for b in range(_NBUF):
        issue_gather(b, b)

    wait_gather(0, 0)
    issue_wb(0, 0)

    def body(t, carry):
        for j in range(_NBUF):
            g = 1 + t * _NBUF + j
            bp = j
            b = (1 + j) % _NBUF
            wait_wb(g - 1, bp)
            issue_gather(g - 1 + _NBUF, bp)
            wait_gather(g, b)
            issue_wb(g, b)
        return carry

    lax.fori_loop(0, (_NCH - _NBUF) // _NBUF, body, 0)

    for g in range(_NCH - _NBUF + 1, _NCH):
        wait_wb(g - 1, (g - 1) % _NBUF)
        wait_gather(g, g % _NBUF)
        issue_wb(g, g % _NBUF)

    wait_wb(_NCH - 1, (_NCH - 1) % _NBUF)


@jax.jit
def kernel(x, table):
    mesh = plsc.VectorSubcoreMesh(core_axis_name="c", subcore_axis_name="s")
    tpad = pl.kernel(
        _transpose_body,
        out_type=jax.ShapeDtypeStruct((_VPAD, 128), jnp.float32),
        mesh=mesh,
        scratch_types=[
            [pltpu.VMEM((_D, 128), jnp.float32) for _ in range(2)],
            [pltpu.VMEM((128, 133), jnp.float32) for _ in range(2)],
            [pltpu.SemaphoreType.DMA for _ in range(2)],
            [pltpu.SemaphoreType.DMA for _ in range(2)],
            pltpu.VMEM((_D, 64), jnp.float32),
        ],
        compiler_params=pltpu.CompilerParams(use_tc_tiling_on_sc=True,
                                             needs_layout_passes=False),
    )(table.T)
    out = pl.kernel(
        _gather_body,
        out_type=jax.ShapeDtypeStruct((_BATCH, _S, 128), jnp.float32),
        mesh=mesh,
        scratch_types=[
            pltpu.VMEM((_BPW,), jnp.int32),
            [pltpu.VMEM((_S, 128), jnp.float32) for _ in range(_NBUF)],
            [pltpu.SemaphoreType.DMA for _ in range(_NBUF)],
            [pltpu.SemaphoreType.DMA for _ in range(_NBUF)],
        ],
        compiler_params=pltpu.CompilerParams(use_tc_tiling_on_sc=True),
    )(x.reshape(-1).astype(jnp.int32), tpad)
    return out[:, :, :_D]


# speedup vs baseline: 1.2216x; 1.0451x over previous
import jax
import jax.numpy as jnp
from jax import lax
from jax.experimental import pallas as pl
from jax.experimental.pallas import tpu as pltpu
from jax.experimental.pallas import tpu_sc as plsc

_V = 1000000
_D = 64
_BATCH = 4096
_S = 200
_NC = 2
_NS = 16
_NW = _NC * _NS

_NTC_FULL = _V // 128
_TPW = 244
_REM = _NTC_FULL - _TPW * _NW
_VPAD = (_NTC_FULL + 1) * 128

_NCH = _BATCH // _NW
_BPW = _NCH * _S
_NBUF = 4


def _transpose_body(tt_hbm, tpad_hbm, stage, outb, sems_i, sems_o, tail_v):
    wid = lax.axis_index("s") * _NC + lax.axis_index("c")
    lo = wid * _TPW

    def start_in(u, b):
        pltpu.async_copy(
            tt_hbm.at[:, pl.ds(128 * (lo + u), 128)], stage[b], sems_i[b])

    def wait_in(u, b):
        pltpu.make_async_copy(
            tt_hbm.at[:, pl.ds(128 * (lo + u), 128)], stage[b], sems_i[b]).wait()

    def start_out(u, b):
        pltpu.async_copy(
            outb[b].at[:, pl.ds(0, 128)],
            tpad_hbm.at[pl.ds(128 * (lo + u), 128)], sems_o[b])

    def wait_out(u, b):
        pltpu.make_async_copy(
            outb[b].at[:, pl.ds(0, 128)],
            tpad_hbm.at[pl.ds(128 * (lo + u), 128)], sems_o[b]).wait()

    jbs = [lax.iota(jnp.int32, 16) + 16 * jb for jb in range(8)]

    def transpose(b, njb):
        @plsc.parallel_loop(0, _D, unroll=4)
        def do_d(d):
            dd = jnp.full((16,), 0, jnp.int32) + d
            vals = [stage[b][d, pl.ds(16 * jb, 16)] for jb in range(njb)]
            for jb in range(njb):
                plsc.store_scatter(outb[b], [jbs[jb], dd], vals[jb])

    start_in(0, 0)
    start_in(1, 1)
    wait_in(0, 0)
    transpose(0, 8)
    start_out(0, 0)
    start_in(2, 0)
    wait_in(1, 1)
    transpose(1, 8)
    start_out(1, 1)

    def body(t, carry):
        for j in range(2):
            u = 2 + t * 2 + j
            b = j
            start_in(u + 1, 1 - b)
            wait_in(u, b)
            wait_out(u - 2, b)
            transpose(b, 8)
            start_out(u, b)
        return carry

    lax.fori_loop(0, (_TPW - 4) // 2, body, 0)

    start_in(243, 1)
    wait_in(242, 0)
    wait_out(240, 0)
    transpose(0, 8)
    start_out(242, 0)
    wait_in(243, 1)
    wait_out(241, 1)
    transpose(1, 8)
    start_out(243, 1)
    wait_out(242, 0)
    wait_out(243, 1)

    @pl.when(wid < _REM)
    def _leftover():
        tc = _NTC_FULL - _REM + wid
        pltpu.sync_copy(tt_hbm.at[:, pl.ds(128 * tc, 128)], stage[0])
        transpose(0, 8)
        pltpu.sync_copy(
            outb[0].at[:, pl.ds(0, 128)], tpad_hbm.at[pl.ds(128 * tc, 128)])

    @pl.when(wid == _NW - 1)
    def _tail():
        pltpu.sync_copy(tt_hbm.at[:, pl.ds(128 * _NTC_FULL, 64)], tail_v)
        @plsc.parallel_loop(0, _D, unroll=4)
        def do_d(d):
            dd = jnp.full((16,), 0, jnp.int32) + d
            vals = [tail_v[d, pl.ds(16 * jb, 16)] for jb in range(4)]
            for jb in range(4):
                plsc.store_scatter(outb[0], [jbs[jb], dd], vals[jb])
        pltpu.sync_copy(
            outb[0].at[pl.ds(0, 64), pl.ds(0, 128)],
            tpad_hbm.at[pl.ds(128 * _NTC_FULL, 64)])


def _gather_body(x_hbm, tpad_hbm, out_hbm, idx_v, rows, sems_g, sems_w):
    wid = lax.axis_index("s") * _NC + lax.axis_index("c")
    base = wid * _NCH

    pltpu.sync_copy(x_hbm.at[pl.ds(base * _S, _BPW)], idx_v)

    def issue_gather(g, b):
        pltpu.async_copy(
            tpad_hbm.at[idx_v.at[pl.ds(g * _S, _S)]], rows[b], sems_g[b])

    def wait_gather(g, b):
        pltpu.make_async_copy(
            tpad_hbm.at[idx_v.at[pl.ds(g * _S, _S)]], rows[b], sems_g[b]).wait()

    def issue_wb(g, b):
        pltpu.async_copy(rows[b], out_hbm.at[base + g], sems_w[b])

    def wait_wb(g, b):
        pltpu.make_async_copy(rows[b], out_hbm.at[base + g], sems_w[b]).wait()

    for b in range(_NBUF):
        issue_gather(b, b)

    wait_gather(0, 0)
    issue_wb(0, 0)

    def body(t, carry):
        for j in range(_NBUF):
            g = 1 + t * _NBUF + j
            bp = j
            b = (1 + j) % _NBUF
            wait_wb(g - 1, bp)
            issue_gather(g - 1 + _NBUF, bp)
            wait_gather(g, b)
            issue_wb(g, b)
        return carry

    lax.fori_loop(0, (_NCH - _NBUF) // _NBUF, body, 0)

    for g in range(_NCH - _NBUF + 1, _NCH):
        wait_wb(g - 1, (g - 1) % _NBUF)
        wait_gather(g, g % _NBUF)
        issue_wb(g, g % _NBUF)

    wait_wb(_NCH - 1, (_NCH - 1) % _NBUF)


@jax.jit
def kernel(x, table):
    mesh = plsc.VectorSubcoreMesh(core_axis_name="c", subcore_axis_name="s")
    tpad = pl.kernel(
        _transpose_body,
        out_type=jax.ShapeDtypeStruct((_VPAD, 128), jnp.float32),
        mesh=mesh,
        scratch_types=[
            [pltpu.VMEM((_D, 128), jnp.float32) for _ in range(2)],
            [pltpu.VMEM((128, 133), jnp.float32) for _ in range(2)],
            [pltpu.SemaphoreType.DMA for _ in range(2)],
            [pltpu.SemaphoreType.DMA for _ in range(2)],
            pltpu.VMEM((_D, 64), jnp.float32),
        ],
        compiler_params=pltpu.CompilerParams(use_tc_tiling_on_sc=True,
                                             needs_layout_passes=False),
    )(table.T)
    out = pl.kernel(
        _gather_body,
        out_type=jax.ShapeDtypeStruct((_BATCH, _S, 128), jnp.float32),
        mesh=mesh,
        scratch_types=[
            pltpu.VMEM((_BPW,), jnp.int32),
            [pltpu.VMEM((_S, 128), jnp.float32) for _ in range(_NBUF)],
            [pltpu.SemaphoreType.DMA for _ in range(_NBUF)],
            [pltpu.SemaphoreType.DMA for _ in range(_NBUF)],
        ],
        compiler_params=pltpu.CompilerParams(use_tc_tiling_on_sc=True),
    )(x.reshape(-1).astype(jnp.int32), tpad)
    return out[:, :, :_D]


# speedup vs baseline: 1.6004x; 1.3101x over previous
import jax
import jax.numpy as jnp
from jax import lax
from jax.experimental import pallas as pl
from jax.experimental.pallas import tpu as pltpu
from jax.experimental.pallas import tpu_sc as plsc

_B = 4096 * 200
_D = 64
_NC = 2
_NS = 16
_NW = _NC * _NS
_BPW = _B // _NW
_C = 256
_NCH = _BPW // _C
_NBUF = 4


def _gather_kernel(x_hbm, table_hbm, out_hbm, idx_v, rows, sems_g, sems_w):
    wid = lax.axis_index("s") * _NC + lax.axis_index("c")
    base = wid * _BPW

    pltpu.sync_copy(x_hbm.at[pl.ds(wid * _NCH, _NCH)], idx_v)

    def issue_gather(g, b):
        pltpu.async_copy(table_hbm.at[idx_v.at[g]], rows[b], sems_g[b])

    def wait_gather(g, b):
        pltpu.make_async_copy(table_hbm.at[idx_v.at[g]], rows[b], sems_g[b]).wait()

    def issue_wb(g, b):
        pltpu.async_copy(rows[b], out_hbm.at[pl.ds(base + g * _C, _C)], sems_w[b])

    def wait_wb(g, b):
        pltpu.make_async_copy(
            rows[b], out_hbm.at[pl.ds(base + g * _C, _C)], sems_w[b]).wait()

    for b in range(_NBUF):
        issue_gather(b, b)

    wait_gather(0, 0)
    issue_wb(0, 0)

    def body(t, carry):
        for j in range(_NBUF):
            g = 1 + t * _NBUF + j
            bp = j
            b = (1 + j) % _NBUF
            wait_wb(g - 1, bp)
            issue_gather(g - 1 + _NBUF, bp)
            wait_gather(g, b)
            issue_wb(g, b)
        return carry

    n_main = (_NCH - _NBUF) // _NBUF
    lax.fori_loop(0, n_main, body, 0)

    for g in range(_NCH - _NBUF + 1, _NCH):
        wait_wb(g - 1, (g - 1) % _NBUF)
        wait_gather(g, g % _NBUF)
        issue_wb(g, g % _NBUF)

    wait_wb(_NCH - 1, (_NCH - 1) % _NBUF)


@jax.jit
def kernel(x, table):
    x_2d = x.reshape(_NW * _NCH, _C).astype(jnp.int32)
    mesh = plsc.VectorSubcoreMesh(core_axis_name="c", subcore_axis_name="s")
    out = pl.kernel(
        _gather_kernel,
        out_type=jax.ShapeDtypeStruct((_B, _D), jnp.float32),
        mesh=mesh,
        scratch_types=[
            pltpu.VMEM((_NCH, _C), jnp.int32),
            [pltpu.VMEM((_C, _D), jnp.float32) for _ in range(_NBUF)],
            [pltpu.SemaphoreType.DMA for _ in range(_NBUF)],
            [pltpu.SemaphoreType.DMA for _ in range(_NBUF)],
        ],
        compiler_params=pltpu.CompilerParams(use_tc_tiling_on_sc=False),
    )(x_2d, table)
    return out.reshape(x.shape[0], x.shape[1], _D)


# speedup vs baseline: 2.1258x; 1.3283x over previous
import jax
import jax.numpy as jnp
from jax import lax
from jax.experimental import pallas as pl
from jax.experimental.pallas import tpu as pltpu
from jax.experimental.pallas import tpu_sc as plsc

_V = 1000000
_D = 64
_BATCH = 4096
_S = 200
_NC = 2
_NS = 16
_NW = _NC * _NS
_NCH = _BATCH // _NW
_BPW = _NCH * _S
_NBUF = 4


def _gather_kernel(x_hbm, table_hbm, out_hbm, idx_v, rows, sems_g, sems_w):
    wid = lax.axis_index("s") * _NC + lax.axis_index("c")
    base = wid * _NCH

    pltpu.sync_copy(x_hbm.at[pl.ds(base * _S, _BPW)], idx_v)

    def issue_gather(g, b):
        pltpu.async_copy(
            table_hbm.at[idx_v.at[pl.ds(g * _S, _S)]], rows[b], sems_g[b])

    def wait_gather(g, b):
        pltpu.make_async_copy(
            table_hbm.at[idx_v.at[pl.ds(g * _S, _S)]], rows[b], sems_g[b]).wait()

    def issue_wb(g, b):
        pltpu.async_copy(
            rows[b], out_hbm.at[base + g, :, pl.ds(0, _D)], sems_w[b])

    def wait_wb(g, b):
        pltpu.make_async_copy(
            rows[b], out_hbm.at[base + g, :, pl.ds(0, _D)], sems_w[b]).wait()

    for b in range(_NBUF):
        issue_gather(b, b)

    wait_gather(0, 0)
    issue_wb(0, 0)

    def body(t, carry):
        for j in range(_NBUF):
            g = 1 + t * _NBUF + j
            bp = j
            b = (1 + j) % _NBUF
            wait_wb(g - 1, bp)
            issue_gather(g - 1 + _NBUF, bp)
            wait_gather(g, b)
            issue_wb(g, b)
        return carry

    lax.fori_loop(0, (_NCH - _NBUF) // _NBUF, body, 0)

    for g in range(_NCH - _NBUF + 1, _NCH):
        wait_wb(g - 1, (g - 1) % _NBUF)
        wait_gather(g, g % _NBUF)
        issue_wb(g, g % _NBUF)

    wait_wb(_NCH - 1, (_NCH - 1) % _NBUF)


@jax.jit
def kernel(x, table):
    mesh = plsc.VectorSubcoreMesh(core_axis_name="c", subcore_axis_name="s")
    out = pl.kernel(
        _gather_kernel,
        out_type=jax.ShapeDtypeStruct((_BATCH, _S, 128), jnp.float32),
        mesh=mesh,
        scratch_types=[
            pltpu.VMEM((_BPW,), jnp.int32),
            [pltpu.VMEM((_S, _D), jnp.float32) for _ in range(_NBUF)],
            [pltpu.SemaphoreType.DMA for _ in range(_NBUF)],
            [pltpu.SemaphoreType.DMA for _ in range(_NBUF)],
        ],
        compiler_params=pltpu.CompilerParams(use_tc_tiling_on_sc=False),
    )(x.reshape(-1).astype(jnp.int32), table)
    return out[:, :, :_D]
